# baseline (speedup 1.0000x reference)
import functools

import jax
import jax.numpy as jnp
from jax import lax
from jax.experimental import pallas as pl
from jax.experimental.pallas import tpu as pltpu
from jax.experimental.pallas import tpu_sc as plsc

N_NODES = 10000
N_EDGES = 320000
D = 128

NC = 2
NS = 16
NW = NC * NS
L = 16

CHUNK = 128
CPW = 80
BLK = 8
N_CHUNKS = NW * CPW
N_EDGES_PAD = N_CHUNKS * CHUNK
N_ACC = 10240
STRIPE = N_ACC // NS
LAST_STRIPE = N_NODES - (NS - 1) * STRIPE


def _lane_broadcast(v, lane):
    return lax.broadcast_in_dim(v[lane], (L,), ())


def _linear_body(x_ref, wt_ref, b_ref, o_ref):
    o_ref[...] = (
        jnp.dot(x_ref[...], wt_ref[...], preferred_element_type=jnp.float32,
                precision=lax.Precision.HIGHEST)
        + b_ref[...]
    )


def _combine_body(p_ref, o_ref):
    o_ref[...] = p_ref[0] + p_ref[1]


def _sc_body(edata_hbm, wts_hbm, support_hbm, out_hbm,
             ed_v, wts_v, rows_v, acc_sh, gsem0, gsem1):
    ci = lax.axis_index("c")
    si = lax.axis_index("s")
    wid = si * NC + ci
    gsem = (gsem0, gsem1)

    zero16 = jnp.zeros((L,), jnp.float32)

    def zrow(r, carry):
        for j in range(D // L):
            rows_v[0, r, j * L:(j + 1) * L] = zero16
        return carry

    lax.fori_loop(0, CHUNK, zrow, 0)
    base = si * STRIPE
    for t in range(STRIPE // CHUNK):
        pltpu.sync_copy(rows_v.at[0], acc_sh.at[pl.ds(base + t * CHUNK, CHUNK)])
    plsc.subcore_barrier()

    base_c = wid * CPW

    def load_block(blk):
        par = lax.rem(blk, 2)
        pltpu.sync_copy(edata_hbm.at[pl.ds(base_c + blk * BLK, BLK)],
                        ed_v.at[par])
        pltpu.sync_copy(wts_hbm.at[pl.ds(base_c + blk * BLK, BLK)],
                        wts_v.at[par])

    def _ed(i, s):
        return ed_v.at[lax.rem(i // BLK, 2), lax.rem(i, BLK), s]

    def fire_gather(i, b):
        pltpu.async_copy(support_hbm.at[_ed(i, 0)], rows_v.at[b], gsem[b])

    def wait_gather(i, b):
        pltpu.make_async_copy(support_hbm.at[_ed(i, 0)], rows_v.at[b],
                              gsem[b]).wait()

    def sync_scatter(i, b):
        pltpu.sync_copy(rows_v.at[b], acc_sh.at[_ed(i, 1)], add=True)

    load_block(0)
    fire_gather(0, 0)

    def halfstep(i, b):
        wait_gather(i, b)

        @pl.when(jnp.logical_and(i + 1 < CPW, lax.rem(i + 1, BLK) == 0))
        def _():
            load_block((i + 1) // BLK)

        @pl.when(i + 1 < CPW)
        def _():
            fire_gather(i + 1, 1 - b)

        def grp(g, gc):
            wv = wts_v[lax.rem(i // BLK, 2), lax.rem(i, BLK), pl.ds(g * L, L)]
            for r in range(L):
                wb = _lane_broadcast(wv, r)
                row = g * L + r
                for j in range(D // L):
                    sl = pl.ds(j * L, L)
                    rows_v[b, row, sl] = rows_v[b, row, sl] * wb
            return gc

        lax.fori_loop(0, CHUNK // L, grp, 0)
        sync_scatter(i, b)

    def pair(p, carry):
        halfstep(2 * p, 0)
        halfstep(2 * p + 1, 1)
        return carry

    lax.fori_loop(0, CPW // 2, pair, 0)
    plsc.subcore_barrier()

    @pl.when(si < NS - 1)
    def _():
        pltpu.sync_copy(acc_sh.at[pl.ds(base, STRIPE)],
                        out_hbm.at[ci, pl.ds(base, STRIPE)])

    @pl.when(si == NS - 1)
    def _():
        pltpu.sync_copy(acc_sh.at[pl.ds(base, LAST_STRIPE)],
                        out_hbm.at[ci, pl.ds(base, LAST_STRIPE)])


_sc_call = functools.partial(
    pl.kernel,
    out_type=jax.ShapeDtypeStruct((NC, N_NODES, D), jnp.float32),
    mesh=plsc.VectorSubcoreMesh(core_axis_name="c", subcore_axis_name="s"),
    scratch_types=[
        pltpu.VMEM((2, BLK, 2, CHUNK), jnp.int32),
        pltpu.VMEM((2, BLK, CHUNK), jnp.float32),
        pltpu.VMEM((2, CHUNK, D), jnp.float32),
        pltpu.VMEM_SHARED((N_ACC, D), jnp.float32),
        pltpu.SemaphoreType.DMA,
        pltpu.SemaphoreType.DMA,
    ],
)(_sc_body)

_ROWS_BLK = 1000


def kernel(x, edge_index, edge_weight, W, b):
    pad = N_EDGES_PAD - N_EDGES
    src = jnp.concatenate([edge_index[0], jnp.zeros((pad,), jnp.int32)])
    dst = jnp.concatenate([edge_index[1], jnp.zeros((pad,), jnp.int32)])
    wpad = jnp.concatenate([edge_weight, jnp.zeros((pad,), jnp.float32)])
    edata = jnp.stack([src.reshape(N_CHUNKS, CHUNK),
                       dst.reshape(N_CHUNKS, CHUNK)], axis=1)
    wts = wpad.reshape(N_CHUNKS, CHUNK)
    wt = W.T
    b2 = b.reshape(1, D)

    support = pl.pallas_call(
        _linear_body,
        grid=(N_NODES // _ROWS_BLK,),
        in_specs=[
            pl.BlockSpec((_ROWS_BLK, D), lambda i: (i, 0)),
            pl.BlockSpec((D, D), lambda i: (0, 0)),
            pl.BlockSpec((1, D), lambda i: (0, 0)),
        ],
        out_specs=pl.BlockSpec((_ROWS_BLK, D), lambda i: (i, 0)),
        out_shape=jax.ShapeDtypeStruct((N_NODES, D), jnp.float32),
    )(x, wt, b2)

    partials = _sc_call(edata, wts, support)

    out = pl.pallas_call(
        _combine_body,
        grid=(N_NODES // _ROWS_BLK,),
        in_specs=[pl.BlockSpec((NC, _ROWS_BLK, D), lambda i: (0, i, 0))],
        out_specs=pl.BlockSpec((_ROWS_BLK, D), lambda i: (i, 0)),
        out_shape=jax.ShapeDtypeStruct((N_NODES, D), jnp.float32),
    )(partials)

    return out


# speedup vs baseline: 1.0009x; 1.0009x over previous
import functools

import jax
import jax.numpy as jnp
from jax import lax
from jax.experimental import pallas as pl
from jax.experimental.pallas import tpu as pltpu
from jax.experimental.pallas import tpu_sc as plsc

N_NODES = 10000
N_EDGES = 320000
D = 128

NC = 2
NS = 16
NW = NC * NS
L = 16

CHUNK = 128
CPW = 80
BLK = 8
N_CHUNKS = NW * CPW
N_EDGES_PAD = N_CHUNKS * CHUNK
N_ACC = 10240
STRIPE = N_ACC // NS
LAST_STRIPE = N_NODES - (NS - 1) * STRIPE


def _lane_broadcast(v, lane):
    return lax.broadcast_in_dim(v[lane], (L,), ())


def _linear_body(x_ref, wt_ref, b_ref, o_ref):
    o_ref[...] = (
        jnp.dot(x_ref[...], wt_ref[...], preferred_element_type=jnp.float32,
                precision=lax.Precision.HIGHEST)
        + b_ref[...]
    )


def _combine_body(p_ref, o_ref):
    o_ref[...] = p_ref[0] + p_ref[1]


def _sc_body(edata_hbm, wts_hbm, support_hbm, out_hbm,
             ed_v, wts_v, rows_v, acc_sh, gsem0, gsem1):
    ci = lax.axis_index("c")
    si = lax.axis_index("s")
    wid = si * NC + ci
    gsem = (gsem0, gsem1)

    zero16 = jnp.zeros((L,), jnp.float32)

    def zrow(r, carry):
        for j in range(D // L):
            rows_v[0, r, j * L:(j + 1) * L] = zero16
        return carry

    lax.fori_loop(0, CHUNK, zrow, 0)
    base = si * STRIPE
    for t in range(STRIPE // CHUNK):
        pltpu.sync_copy(rows_v.at[0], acc_sh.at[pl.ds(base + t * CHUNK, CHUNK)])
    plsc.subcore_barrier()

    base_c = wid * CPW

    def load_block(blk):
        par = blk & 1
        pltpu.sync_copy(edata_hbm.at[pl.ds(base_c + blk * BLK, BLK)],
                        ed_v.at[par])
        pltpu.sync_copy(wts_hbm.at[pl.ds(base_c + blk * BLK, BLK)],
                        wts_v.at[par])

    def _ed(i, s):
        return ed_v.at[(i >> 3) & 1, i & (BLK - 1), s]

    def fire_gather(i, b):
        pltpu.async_copy(support_hbm.at[_ed(i, 0)], rows_v.at[b], gsem[b])

    def wait_gather(i, b):
        pltpu.make_async_copy(support_hbm.at[_ed(i, 0)], rows_v.at[b],
                              gsem[b]).wait()

    def sync_scatter(i, b):
        pltpu.sync_copy(rows_v.at[b], acc_sh.at[_ed(i, 1)], add=True)

    load_block(0)
    fire_gather(0, 0)

    def halfstep(i, b):
        wait_gather(i, b)

        @pl.when(jnp.logical_and(i + 1 < CPW, ((i + 1) & (BLK - 1)) == 0))
        def _():
            load_block((i + 1) >> 3)

        @pl.when(i + 1 < CPW)
        def _():
            fire_gather(i + 1, 1 - b)

        par_i = (i >> 3) & 1
        jj_i = i & (BLK - 1)

        def grp(g, gc):
            wv = wts_v[par_i, jj_i, pl.ds(g * L, L)]
            for r in range(L):
                wb = _lane_broadcast(wv, r)
                row = g * L + r
                for j in range(D // L):
                    sl = pl.ds(j * L, L)
                    rows_v[b, row, sl] = rows_v[b, row, sl] * wb
            return gc

        lax.fori_loop(0, CHUNK // L, grp, 0)
        sync_scatter(i, b)

    def pair(p, carry):
        halfstep(2 * p, 0)
        halfstep(2 * p + 1, 1)
        return carry

    lax.fori_loop(0, CPW // 2, pair, 0)
    plsc.subcore_barrier()

    @pl.when(si < NS - 1)
    def _():
        pltpu.sync_copy(acc_sh.at[pl.ds(base, STRIPE)],
                        out_hbm.at[ci, pl.ds(base, STRIPE)])

    @pl.when(si == NS - 1)
    def _():
        pltpu.sync_copy(acc_sh.at[pl.ds(base, LAST_STRIPE)],
                        out_hbm.at[ci, pl.ds(base, LAST_STRIPE)])


_sc_call = functools.partial(
    pl.kernel,
    out_type=jax.ShapeDtypeStruct((NC, N_NODES, D), jnp.float32),
    mesh=plsc.VectorSubcoreMesh(core_axis_name="c", subcore_axis_name="s"),
    scratch_types=[
        pltpu.VMEM((2, BLK, 2, CHUNK), jnp.int32),
        pltpu.VMEM((2, BLK, CHUNK), jnp.float32),
        pltpu.VMEM((2, CHUNK, D), jnp.float32),
        pltpu.VMEM_SHARED((N_ACC, D), jnp.float32),
        pltpu.SemaphoreType.DMA,
        pltpu.SemaphoreType.DMA,
    ],
)(_sc_body)

_ROWS_BLK = 1000


def kernel(x, edge_index, edge_weight, W, b):
    pad = N_EDGES_PAD - N_EDGES
    src = jnp.concatenate([edge_index[0], jnp.zeros((pad,), jnp.int32)])
    dst = jnp.concatenate([edge_index[1], jnp.zeros((pad,), jnp.int32)])
    wpad = jnp.concatenate([edge_weight, jnp.zeros((pad,), jnp.float32)])
    edata = jnp.stack([src.reshape(N_CHUNKS, CHUNK),
                       dst.reshape(N_CHUNKS, CHUNK)], axis=1)
    wts = wpad.reshape(N_CHUNKS, CHUNK)
    wt = W.T
    b2 = b.reshape(1, D)

    support = pl.pallas_call(
        _linear_body,
        grid=(N_NODES // _ROWS_BLK,),
        in_specs=[
            pl.BlockSpec((_ROWS_BLK, D), lambda i: (i, 0)),
            pl.BlockSpec((D, D), lambda i: (0, 0)),
            pl.BlockSpec((1, D), lambda i: (0, 0)),
        ],
        out_specs=pl.BlockSpec((_ROWS_BLK, D), lambda i: (i, 0)),
        out_shape=jax.ShapeDtypeStruct((N_NODES, D), jnp.float32),
    )(x, wt, b2)

    partials = _sc_call(edata, wts, support)

    out = pl.pallas_call(
        _combine_body,
        grid=(N_NODES // _ROWS_BLK,),
        in_specs=[pl.BlockSpec((NC, _ROWS_BLK, D), lambda i: (0, i, 0))],
        out_specs=pl.BlockSpec((_ROWS_BLK, D), lambda i: (i, 0)),
        out_shape=jax.ShapeDtypeStruct((N_NODES, D), jnp.float32),
    )(partials)

    return out
